# baseline (speedup 1.0000x reference)
import jax
import jax.numpy as jnp
from jax.experimental import pallas as pl
from jax.experimental.pallas import tpu as pltpu

NUM_EXPERTS = 64
TOP_K = 8
BLOCK_T = 2048
BLOCK_K = 1024
_INT_MIN = -(2**31)


def _router_block(x_ref, w_ref, wout_ref, iout_ref, lout_ref, acc_ref):
    kstep = pl.program_id(1)
    nk = pl.num_programs(1)

    part = jax.lax.dot_general(
        w_ref[...], x_ref[...],
        (((1,), (1,)), ((), ())),
        preferred_element_type=jnp.float32,
    )

    @pl.when(kstep == 0)
    def _init():
        acc_ref[...] = part

    @pl.when(kstep != 0)
    def _accum():
        acc_ref[...] += part

    @pl.when(kstep == nk - 1)
    def _epilogue():
        logits_t = acc_ref[...]
        lout_ref[...] = logits_t.T

        bt = logits_t.shape[1]
        bits = jax.lax.bitcast_convert_type(logits_t, jnp.int32)
        okey = jnp.where(bits >= 0, bits, jnp.int32(_INT_MIN) - bits)
        iota = jax.lax.broadcasted_iota(jnp.int32, (NUM_EXPERTS, bt), 0)
        key = (okey & jnp.int32(~63)) | (jnp.int32(63) - iota)

        tops = []
        for _ in range(TOP_K):
            mk = jnp.max(key, axis=0, keepdims=True)
            tops.append(mk)
            key = jnp.where(key == mk, jnp.int32(_INT_MIN), key)

        top = jnp.concatenate(tops, axis=0)
        idx = jnp.int32(63) - (top & jnp.int32(63))
        vkey = top & jnp.int32(~63)
        vbits = jnp.where(vkey >= 0, vkey, jnp.int32(_INT_MIN) - vkey)
        lsel = jax.lax.bitcast_convert_type(vbits, jnp.float32)
        e = jnp.exp(lsel - lsel[:1, :])
        wsel = e / jnp.sum(e, axis=0, keepdims=True)
        wout_ref[...] = wsel.T
        iout_ref[...] = idx.T


def kernel(x, W):
    n_tokens, d_model = x.shape
    grid = (n_tokens // BLOCK_T, d_model // BLOCK_K)
    out_shapes = (
        jax.ShapeDtypeStruct((n_tokens, TOP_K), jnp.float32),
        jax.ShapeDtypeStruct((n_tokens, TOP_K), jnp.int32),
        jax.ShapeDtypeStruct((n_tokens, NUM_EXPERTS), jnp.float32),
    )
    return pl.pallas_call(
        _router_block,
        grid=grid,
        in_specs=[
            pl.BlockSpec((BLOCK_T, BLOCK_K), lambda i, k: (i, k)),
            pl.BlockSpec((NUM_EXPERTS, BLOCK_K), lambda i, k: (0, k)),
        ],
        out_specs=(
            pl.BlockSpec((BLOCK_T, TOP_K), lambda i, k: (i, 0)),
            pl.BlockSpec((BLOCK_T, TOP_K), lambda i, k: (i, 0)),
            pl.BlockSpec((BLOCK_T, NUM_EXPERTS), lambda i, k: (i, 0)),
        ),
        out_shape=out_shapes,
        scratch_shapes=[pltpu.VMEM((NUM_EXPERTS, BLOCK_T), jnp.float32)],
        compiler_params=pltpu.CompilerParams(
            dimension_semantics=("arbitrary", "arbitrary"),
        ),
    )(x, W)
